# Initial kernel scaffold; baseline (speedup 1.0000x reference)
import jax
import jax.numpy as jnp
from jax.experimental import pallas as pl


def kernel(input, boxes, masks_roi, score_mask, lin1_w, lin1_b, lin2_w, lin2_b, conv1_w, conv1_b, conv2_w, conv2_b, ln_w, ln_b):
    raise NotImplementedError("write your pallas kernel here")


# speedup vs baseline: 39.5242x; 39.5242x over previous
import jax
import jax.numpy as jnp
from jax.experimental import pallas as pl

_B = 2
_NUM = 256
_CF = 128
_H = 4
_K = 32
_G = 4
_GW = _CF // _G
_HI = jax.lax.Precision.HIGHEST


def _topk_union_update(S, hit, iota_j):
    def body(_, carry):
        work, h = carry
        m = jnp.max(work, axis=1, keepdims=True)
        cand = jnp.where(work == m, iota_j, 2 * _NUM)
        fidx = jnp.min(cand, axis=1, keepdims=True)
        sel = cand == fidx
        h = jnp.maximum(h, jnp.max(jnp.where(sel, 1.0, 0.0), axis=0, keepdims=True))
        work = jnp.where(sel, -1.0, work)
        return work, h

    _, hit = jax.lax.fori_loop(0, _K, body, (S, hit))
    return hit


def _stage(xs, boxes, boxesTs, rois, fcols, waT, was, wcx, wcs, brow, Wbd, cb,
           iota_j, eye):
    xTs, Ss = [], []
    hit = jnp.zeros((1, _NUM), jnp.float32)
    for b in range(_B):
        x = xs[b]
        xT = jnp.transpose(x, (1, 0))
        inv_col = 1.0 / jnp.maximum(
            jnp.sqrt(jnp.sum(x * x, axis=1, keepdims=True)), 1e-8)
        inv_row = 1.0 / jnp.maximum(
            jnp.sqrt(jnp.sum(xT * xT, axis=0, keepdims=True)), 1e-8)
        G = jnp.dot(x, xT, preferred_element_type=jnp.float32, precision=_HI)
        S = jnp.maximum(G * inv_col * inv_row, 0.0)
        hit = _topk_union_update(S, hit, iota_j)
        xTs.append(xT)
    colmask = hit

    outs = []
    for b in range(_B):
        x, xT = xs[b], xTs[b]
        aT = (jnp.dot(waT, xT, preferred_element_type=jnp.float32, precision=_HI)
              + jnp.dot(was, boxesTs[b], preferred_element_type=jnp.float32,
                        precision=_HI))
        cC = (jnp.dot(x, wcx, preferred_element_type=jnp.float32, precision=_HI)
              + jnp.dot(boxes[b], wcs, preferred_element_type=jnp.float32,
                        precision=_HI) + brow)
        conv = jnp.maximum(
            jnp.dot(x, Wbd, preferred_element_type=jnp.float32, precision=_HI)
            + cb, 0.0)
        roi = rois[b]
        fcol = fcols[b]
        pieces = []
        for h in range(_H):
            L = cC[:, h:h + 1] + aT[h:h + 1, :]
            P = jax.nn.sigmoid(L)
            M = (P * roi * colmask + fcol * eye) * 0.25
            pieces.append(
                jnp.dot(M, conv[:, h * _GW:(h + 1) * _GW],
                        preferred_element_type=jnp.float32, precision=_HI))
        outs.append(conv + jnp.concatenate(pieces, axis=1))
    return outs


def _body(x_ref, boxes_ref, boxesT_ref, roi_ref, smrow_ref, smcol_ref,
          wa1T_ref, wa1s_ref, wc1x_ref, wc1s_ref, b1_ref,
          wa2T_ref, wa2s_ref, wc2x_ref, wc2s_ref, b2_ref,
          Wbd1_ref, cb1_ref, Wbd2_ref, cb2_ref, lnw_ref, lnb_ref, out_ref):
    iota_j = jax.lax.broadcasted_iota(jnp.int32, (_NUM, _NUM), 1)
    iota_i = jax.lax.broadcasted_iota(jnp.int32, (_NUM, _NUM), 0)
    eye = jnp.where(iota_i == iota_j, 1.0, 0.0)

    xs = [x_ref[b] for b in range(_B)]
    boxes = [boxes_ref[b] for b in range(_B)]
    boxesTs = [boxesT_ref[b] for b in range(_B)]
    rois = [roi_ref[b] * smrow_ref[b] for b in range(_B)]
    fcols = [jnp.where(smcol_ref[b] == 0.0, 1.0, 0.0) for b in range(_B)]

    mids = _stage(xs, boxes, boxesTs, rois, fcols,
                  wa1T_ref[...], wa1s_ref[...], wc1x_ref[...], wc1s_ref[...],
                  b1_ref[...], Wbd1_ref[...], cb1_ref[...], iota_j, eye)
    outs = _stage(mids, boxes, boxesTs, rois, fcols,
                  wa2T_ref[...], wa2s_ref[...], wc2x_ref[...], wc2s_ref[...],
                  b2_ref[...], Wbd2_ref[...], cb2_ref[...], iota_j, eye)

    lnw = lnw_ref[...]
    lnb = lnb_ref[...]
    for b in range(_B):
        v = outs[b]
        mu = jnp.mean(v, axis=1, keepdims=True)
        d = v - mu
        var = jnp.mean(d * d, axis=1, keepdims=True)
        out_ref[b] = d * jax.lax.rsqrt(var + 1e-6) * lnw + lnb


def _blockdiag(w):
    z = jnp.zeros((_CF, _CF), jnp.float32)
    for g in range(_G):
        z = z.at[g * _GW:(g + 1) * _GW, g * _GW:(g + 1) * _GW].set(
            jnp.transpose(w[g * _GW:(g + 1) * _GW, :]))
    return z


def kernel(input, boxes, masks_roi, score_mask, lin1_w, lin1_b, lin2_w, lin2_b,
           conv1_w, conv1_b, conv2_w, conv2_b, ln_w, ln_b):
    x = input.astype(jnp.float32)
    boxesT = jnp.swapaxes(boxes, 1, 2)
    smrow = score_mask[:, None, :]
    smcol = score_mask[:, :, None]

    def split_lin(w):
        waT = w[:, :_CF]
        was = w[:, 2 * _CF:2 * _CF + 2]
        wcx = jnp.transpose(w[:, _CF:2 * _CF])
        wcs = jnp.transpose(w[:, 2 * _CF + 2:2 * _CF + 4])
        return waT, was, wcx, wcs

    wa1T, wa1s, wc1x, wc1s = split_lin(lin1_w)
    wa2T, wa2s, wc2x, wc2s = split_lin(lin2_w)
    b1 = lin1_b[None, :]
    b2 = lin2_b[None, :]
    Wbd1 = _blockdiag(conv1_w)
    Wbd2 = _blockdiag(conv2_w)
    cb1 = conv1_b[None, :]
    cb2 = conv2_b[None, :]
    lnw = ln_w[None, :]
    lnb = ln_b[None, :]

    return pl.pallas_call(
        _body,
        out_shape=jax.ShapeDtypeStruct((_B, _NUM, _CF), jnp.float32),
    )(x, boxes, boxesT, masks_roi, smrow, smcol,
      wa1T, wa1s, wc1x, wc1s, b1,
      wa2T, wa2s, wc2x, wc2s, b2,
      Wbd1, cb1, Wbd2, cb2, lnw, lnb)


# speedup vs baseline: 49.4767x; 1.2518x over previous
import jax
import jax.numpy as jnp
from jax.experimental import pallas as pl

_B = 2
_NUM = 256
_CF = 128
_H = 4
_K = 32
_G = 4
_GW = _CF // _G
_HI = jax.lax.Precision.HIGHEST


def _topk_union(S):
    iota_j = jax.lax.broadcasted_iota(jnp.int32, S.shape, 1)

    def body(_, work):
        m = jnp.max(work, axis=1, keepdims=True)
        cand = jnp.where(work == m, iota_j, 2 * _NUM)
        fidx = jnp.min(cand, axis=1, keepdims=True)
        return jnp.where(cand == fidx, -1.0, work)

    work = jax.lax.fori_loop(0, _K, body, S)
    return jnp.max(jnp.where(work == -1.0, 1.0, 0.0), axis=0, keepdims=True)


def _stage(xs, boxes, boxesTs, rois, fcols, waT, was, wcx, wcs, brow, Wbd, cb,
           iota_j, eye):
    xTs, Ss = [], []
    for b in range(_B):
        x = xs[b]
        xT = jnp.transpose(x, (1, 0))
        inv_col = 1.0 / jnp.maximum(
            jnp.sqrt(jnp.sum(x * x, axis=1, keepdims=True)), 1e-8)
        inv_row = 1.0 / jnp.maximum(
            jnp.sqrt(jnp.sum(xT * xT, axis=0, keepdims=True)), 1e-8)
        G = jnp.dot(x, xT, preferred_element_type=jnp.float32, precision=_HI)
        Ss.append(jnp.maximum(G * inv_col * inv_row, 0.0))
        xTs.append(xT)
    colmask = _topk_union(jnp.concatenate(Ss, axis=0))

    outs = []
    for b in range(_B):
        x, xT = xs[b], xTs[b]
        aT = (jnp.dot(waT, xT, preferred_element_type=jnp.float32, precision=_HI)
              + jnp.dot(was, boxesTs[b], preferred_element_type=jnp.float32,
                        precision=_HI))
        cC = (jnp.dot(x, wcx, preferred_element_type=jnp.float32, precision=_HI)
              + jnp.dot(boxes[b], wcs, preferred_element_type=jnp.float32,
                        precision=_HI) + brow)
        conv = jnp.maximum(
            jnp.dot(x, Wbd, preferred_element_type=jnp.float32, precision=_HI)
            + cb, 0.0)
        roi = rois[b]
        fcol = fcols[b]
        pieces = []
        for h in range(_H):
            L = cC[:, h:h + 1] + aT[h:h + 1, :]
            P = jax.nn.sigmoid(L)
            M = (P * roi * colmask + fcol * eye) * 0.25
            pieces.append(
                jnp.dot(M, conv[:, h * _GW:(h + 1) * _GW],
                        preferred_element_type=jnp.float32, precision=_HI))
        outs.append(conv + jnp.concatenate(pieces, axis=1))
    return outs


def _body(x_ref, boxes_ref, boxesT_ref, roi_ref, smrow_ref, smcol_ref,
          wa1T_ref, wa1s_ref, wc1x_ref, wc1s_ref, b1_ref,
          wa2T_ref, wa2s_ref, wc2x_ref, wc2s_ref, b2_ref,
          Wbd1_ref, cb1_ref, Wbd2_ref, cb2_ref, lnw_ref, lnb_ref, out_ref):
    iota_j = jax.lax.broadcasted_iota(jnp.int32, (_NUM, _NUM), 1)
    iota_i = jax.lax.broadcasted_iota(jnp.int32, (_NUM, _NUM), 0)
    eye = jnp.where(iota_i == iota_j, 1.0, 0.0)

    xs = [x_ref[b] for b in range(_B)]
    boxes = [boxes_ref[b] for b in range(_B)]
    boxesTs = [boxesT_ref[b] for b in range(_B)]
    rois = [roi_ref[b] * smrow_ref[b] for b in range(_B)]
    fcols = [jnp.where(smcol_ref[b] == 0.0, 1.0, 0.0) for b in range(_B)]

    mids = _stage(xs, boxes, boxesTs, rois, fcols,
                  wa1T_ref[...], wa1s_ref[...], wc1x_ref[...], wc1s_ref[...],
                  b1_ref[...], Wbd1_ref[...], cb1_ref[...], iota_j, eye)
    outs = _stage(mids, boxes, boxesTs, rois, fcols,
                  wa2T_ref[...], wa2s_ref[...], wc2x_ref[...], wc2s_ref[...],
                  b2_ref[...], Wbd2_ref[...], cb2_ref[...], iota_j, eye)

    lnw = lnw_ref[...]
    lnb = lnb_ref[...]
    for b in range(_B):
        v = outs[b]
        mu = jnp.mean(v, axis=1, keepdims=True)
        d = v - mu
        var = jnp.mean(d * d, axis=1, keepdims=True)
        out_ref[b] = d * jax.lax.rsqrt(var + 1e-6) * lnw + lnb


def _blockdiag(w):
    z = jnp.zeros((_CF, _CF), jnp.float32)
    for g in range(_G):
        z = z.at[g * _GW:(g + 1) * _GW, g * _GW:(g + 1) * _GW].set(
            jnp.transpose(w[g * _GW:(g + 1) * _GW, :]))
    return z


def kernel(input, boxes, masks_roi, score_mask, lin1_w, lin1_b, lin2_w, lin2_b,
           conv1_w, conv1_b, conv2_w, conv2_b, ln_w, ln_b):
    x = input.astype(jnp.float32)
    boxesT = jnp.swapaxes(boxes, 1, 2)
    smrow = score_mask[:, None, :]
    smcol = score_mask[:, :, None]

    def split_lin(w):
        waT = w[:, :_CF]
        was = w[:, 2 * _CF:2 * _CF + 2]
        wcx = jnp.transpose(w[:, _CF:2 * _CF])
        wcs = jnp.transpose(w[:, 2 * _CF + 2:2 * _CF + 4])
        return waT, was, wcx, wcs

    wa1T, wa1s, wc1x, wc1s = split_lin(lin1_w)
    wa2T, wa2s, wc2x, wc2s = split_lin(lin2_w)
    b1 = lin1_b[None, :]
    b2 = lin2_b[None, :]
    Wbd1 = _blockdiag(conv1_w)
    Wbd2 = _blockdiag(conv2_w)
    cb1 = conv1_b[None, :]
    cb2 = conv2_b[None, :]
    lnw = ln_w[None, :]
    lnb = ln_b[None, :]

    return pl.pallas_call(
        _body,
        out_shape=jax.ShapeDtypeStruct((_B, _NUM, _CF), jnp.float32),
    )(x, boxes, boxesT, masks_roi, smrow, smcol,
      wa1T, wa1s, wc1x, wc1s, b1,
      wa2T, wa2s, wc2x, wc2s, b2,
      Wbd1, cb1, Wbd2, cb2, lnw, lnb)


# speedup vs baseline: 55.2932x; 1.1176x over previous
import jax
import jax.numpy as jnp
from jax.experimental import pallas as pl

_B = 2
_NUM = 256
_CF = 128
_H = 4
_K = 32
_G = 4
_GW = _CF // _G
_HI = jax.lax.Precision.DEFAULT
_MED = jax.lax.Precision.DEFAULT


def _topk_union(S):
    iota_j = jax.lax.broadcasted_iota(jnp.int32, S.shape, 1)

    def body(_, work):
        m = jnp.max(work, axis=1, keepdims=True)
        cand = jnp.where(work == m, iota_j, 2 * _NUM)
        fidx = jnp.min(cand, axis=1, keepdims=True)
        return jnp.where(cand == fidx, -1.0, work)

    work = jax.lax.fori_loop(0, _K, body, S)
    return jnp.max(jnp.where(work == -1.0, 1.0, 0.0), axis=0, keepdims=True)


def _stage(xs, boxes, boxesTs, rois, fcols, waT, was, wcx, wcs, brow, Wbd, cb,
           iota_j, eye):
    xTs, Ss = [], []
    for b in range(_B):
        x = xs[b]
        xT = jnp.transpose(x, (1, 0))
        inv_col = 1.0 / jnp.maximum(
            jnp.sqrt(jnp.sum(x * x, axis=1, keepdims=True)), 1e-8)
        inv_row = 1.0 / jnp.maximum(
            jnp.sqrt(jnp.sum(xT * xT, axis=0, keepdims=True)), 1e-8)
        G = jnp.dot(x, xT, preferred_element_type=jnp.float32, precision=_HI)
        Ss.append(jnp.maximum(G * inv_col * inv_row, 0.0))
        xTs.append(xT)
    colmask = _topk_union(jnp.concatenate(Ss, axis=0))

    outs = []
    for b in range(_B):
        x, xT = xs[b], xTs[b]
        aT = (jnp.dot(waT, xT, preferred_element_type=jnp.float32, precision=_HI)
              + jnp.dot(was, boxesTs[b], preferred_element_type=jnp.float32,
                        precision=_HI))
        cC = (jnp.dot(x, wcx, preferred_element_type=jnp.float32, precision=_HI)
              + jnp.dot(boxes[b], wcs, preferred_element_type=jnp.float32,
                        precision=_HI) + brow)
        conv = jnp.maximum(
            jnp.dot(x, Wbd, preferred_element_type=jnp.float32, precision=_MED)
            + cb, 0.0)
        roi = rois[b]
        fcol = fcols[b]
        pieces = []
        for h in range(_H):
            L = cC[:, h:h + 1] + aT[h:h + 1, :]
            P = jax.nn.sigmoid(L)
            M = (P * roi * colmask + fcol * eye) * 0.25
            pieces.append(
                jnp.dot(M, conv[:, h * _GW:(h + 1) * _GW],
                        preferred_element_type=jnp.float32, precision=_MED))
        outs.append(conv + jnp.concatenate(pieces, axis=1))
    return outs


def _body(x_ref, boxes_ref, boxesT_ref, roi_ref, smrow_ref, smcol_ref,
          wa1T_ref, wa1s_ref, wc1x_ref, wc1s_ref, b1_ref,
          wa2T_ref, wa2s_ref, wc2x_ref, wc2s_ref, b2_ref,
          Wbd1_ref, cb1_ref, Wbd2_ref, cb2_ref, lnw_ref, lnb_ref, out_ref):
    iota_j = jax.lax.broadcasted_iota(jnp.int32, (_NUM, _NUM), 1)
    iota_i = jax.lax.broadcasted_iota(jnp.int32, (_NUM, _NUM), 0)
    eye = jnp.where(iota_i == iota_j, 1.0, 0.0)

    xs = [x_ref[b] for b in range(_B)]
    boxes = [boxes_ref[b] for b in range(_B)]
    boxesTs = [boxesT_ref[b] for b in range(_B)]
    rois = [roi_ref[b] * smrow_ref[b] for b in range(_B)]
    fcols = [jnp.where(smcol_ref[b] == 0.0, 1.0, 0.0) for b in range(_B)]

    mids = _stage(xs, boxes, boxesTs, rois, fcols,
                  wa1T_ref[...], wa1s_ref[...], wc1x_ref[...], wc1s_ref[...],
                  b1_ref[...], Wbd1_ref[...], cb1_ref[...], iota_j, eye)
    outs = _stage(mids, boxes, boxesTs, rois, fcols,
                  wa2T_ref[...], wa2s_ref[...], wc2x_ref[...], wc2s_ref[...],
                  b2_ref[...], Wbd2_ref[...], cb2_ref[...], iota_j, eye)

    lnw = lnw_ref[...]
    lnb = lnb_ref[...]
    for b in range(_B):
        v = outs[b]
        mu = jnp.mean(v, axis=1, keepdims=True)
        d = v - mu
        var = jnp.mean(d * d, axis=1, keepdims=True)
        out_ref[b] = d * jax.lax.rsqrt(var + 1e-6) * lnw + lnb


def _blockdiag(w):
    z = jnp.zeros((_CF, _CF), jnp.float32)
    for g in range(_G):
        z = z.at[g * _GW:(g + 1) * _GW, g * _GW:(g + 1) * _GW].set(
            jnp.transpose(w[g * _GW:(g + 1) * _GW, :]))
    return z


def kernel(input, boxes, masks_roi, score_mask, lin1_w, lin1_b, lin2_w, lin2_b,
           conv1_w, conv1_b, conv2_w, conv2_b, ln_w, ln_b):
    x = input.astype(jnp.float32)
    boxesT = jnp.swapaxes(boxes, 1, 2)
    smrow = score_mask[:, None, :]
    smcol = score_mask[:, :, None]

    def split_lin(w):
        waT = w[:, :_CF]
        was = w[:, 2 * _CF:2 * _CF + 2]
        wcx = jnp.transpose(w[:, _CF:2 * _CF])
        wcs = jnp.transpose(w[:, 2 * _CF + 2:2 * _CF + 4])
        return waT, was, wcx, wcs

    wa1T, wa1s, wc1x, wc1s = split_lin(lin1_w)
    wa2T, wa2s, wc2x, wc2s = split_lin(lin2_w)
    b1 = lin1_b[None, :]
    b2 = lin2_b[None, :]
    Wbd1 = _blockdiag(conv1_w)
    Wbd2 = _blockdiag(conv2_w)
    cb1 = conv1_b[None, :]
    cb2 = conv2_b[None, :]
    lnw = ln_w[None, :]
    lnb = ln_b[None, :]

    return pl.pallas_call(
        _body,
        out_shape=jax.ShapeDtypeStruct((_B, _NUM, _CF), jnp.float32),
    )(x, boxes, boxesT, masks_roi, smrow, smcol,
      wa1T, wa1s, wc1x, wc1s, b1,
      wa2T, wa2s, wc2x, wc2s, b2,
      Wbd1, cb1, Wbd2, cb2, lnw, lnb)
